# baseline (speedup 1.0000x reference)
import jax
import jax.numpy as jnp
from jax.experimental import pallas as pl
from jax.experimental.pallas import tpu as pltpu

_IN = 2
_H = 32
_OUT = 3
_PACK = 128
_NT = _PACK // 4
_OMEGA = 30.0
_TB = 256

_INV_PI = 0.31830987334251404
_PI_HI = 3.140625
_PI_MID = 0.0009676535846665502
_S1 = 1.0
_S2 = -0.16666658222675323
_S3 = 0.008333050645887852
_S4 = -0.00019809044897556305
_S5 = 2.6051632175949635e-06


def _relu_sin(z):
    n = jnp.rint(z * _INV_PI)
    r = (z - n * _PI_HI) - n * _PI_MID
    q = r * r
    p = _S4 + q * _S5
    p = _S3 + q * p
    p = _S2 + q * p
    p = _S1 + q * p
    s = r * p
    sb = jax.lax.shift_left(jnp.bitwise_and(n.astype(jnp.int32), 1), 31)
    s = jax.lax.bitcast_convert_type(
        jax.lax.bitcast_convert_type(s, jnp.int32) ^ sb, jnp.float32)
    return jnp.maximum(s, 0.0)


def _mlp_kernel(x_ref, w0_ref, b0_ref, w1_ref, b1_ref, wh_ref, bh_ref, o_ref):
    z0 = jnp.dot(x_ref[...], w0_ref[...], preferred_element_type=jnp.float32)
    h0 = _relu_sin(z0 + b0_ref[...])
    parts = []
    for g in range(_NT):
        z1g = jnp.dot(h0[:, 128 * g:128 * (g + 1)], w1_ref[...],
                      preferred_element_type=jnp.float32)
        parts.append(_relu_sin(z1g + b1_ref[...]))
    h1 = jnp.concatenate(parts, axis=1)
    z2 = jnp.dot(h1, wh_ref[...], preferred_element_type=jnp.float32)
    o_ref[...] = z2 + bh_ref[...]


def _round_bf16(a):
    u = jax.lax.bitcast_convert_type(a.astype(jnp.float32), jnp.uint32)
    u = (u + jnp.uint32(0x7FFF) + ((u >> 16) & jnp.uint32(1))) & jnp.uint32(0xFFFF0000)
    return jax.lax.bitcast_convert_type(u, jnp.float32)


@jax.jit
def _run(x, w0, b0, w1, b1, wh, bh):
    B, S, D = x.shape
    M = B * S
    R = M // _PACK

    w0b = jnp.kron(jnp.eye(_PACK, dtype=jnp.float32),
                   (_OMEGA * w0).astype(jnp.float32))
    b0r = jnp.tile(_round_bf16(_OMEGA * b0), (1, _PACK))
    w1b = jnp.kron(jnp.eye(4, dtype=jnp.float32),
                   (_OMEGA * w1).astype(jnp.float32))
    b1r = jnp.tile(_round_bf16(_OMEGA * b1), (1, 4))
    whb = jnp.kron(jnp.eye(_PACK, dtype=jnp.float32),
                   wh.astype(jnp.float32))
    bhr = jnp.tile(_round_bf16(bh), (1, _PACK))

    x128 = x.reshape(R, _PACK * _IN)

    grid = (R // _TB,)
    out = pl.pallas_call(
        _mlp_kernel,
        out_shape=jax.ShapeDtypeStruct((R, _PACK * _OUT), jnp.float32),
        grid=grid,
        in_specs=[
            pl.BlockSpec((_TB, _PACK * _IN), lambda i: (i, 0)),
            pl.BlockSpec((_PACK * _IN, _PACK * _H), lambda i: (0, 0)),
            pl.BlockSpec((1, _PACK * _H), lambda i: (0, 0)),
            pl.BlockSpec((4 * _H, 4 * _H), lambda i: (0, 0)),
            pl.BlockSpec((1, 4 * _H), lambda i: (0, 0)),
            pl.BlockSpec((_PACK * _H, _PACK * _OUT), lambda i: (0, 0)),
            pl.BlockSpec((1, _PACK * _OUT), lambda i: (0, 0)),
        ],
        out_specs=pl.BlockSpec((_TB, _PACK * _OUT), lambda i: (i, 0)),
        compiler_params=pltpu.CompilerParams(
            dimension_semantics=("parallel",),
            vmem_limit_bytes=100 * 1024 * 1024,
        ),
        cost_estimate=pl.CostEstimate(
            flops=2 * M * ((_IN + 1) * _H + (_H + 1) * _H + (_H + 1) * _OUT),
            transcendentals=0,
            bytes_accessed=(_IN + _OUT) * 4 * M,
        ),
    )(x128, w0b, b0r, w1b, b1r, whb, bhr)

    return out.reshape(B, S, _OUT)


def kernel(x, w0, b0, w1, b1, wh, bh):
    return _run(x, w0, b0, w1, b1, wh, bh)


# speedup vs baseline: 8.7018x; 8.7018x over previous
import jax
import jax.numpy as jnp
from jax.experimental import pallas as pl
from jax.experimental.pallas import tpu as pltpu

_IN = 2
_H = 32
_OUT = 3
_OMEGA = 30.0
_TM = 32768

_INV_PI = 0.31830987334251404
_PI_HI = 3.140625
_PI_MID = 0.0009676535846665502
_S2 = -0.16666658222675323
_S3 = 0.008333050645887852
_S4 = -0.00019809044897556305
_S5 = 2.6051632175949635e-06


def _relu_sin(z):
    n = jnp.rint(z * _INV_PI)
    r = (z - n * _PI_HI) - n * _PI_MID
    q = r * r
    p = _S4 + q * _S5
    p = _S3 + q * p
    p = _S2 + q * p
    s = r + (r * q) * p
    sb = jax.lax.shift_left(jnp.bitwise_and(n.astype(jnp.int32), 1), 31)
    s = jax.lax.bitcast_convert_type(
        jax.lax.bitcast_convert_type(s, jnp.int32) ^ sb, jnp.float32)
    return jnp.maximum(s, 0.0)


def _fm_kernel(x_ref, w0_ref, b0_ref, w1_ref, b1_ref, wh_ref, bh_ref, o_ref):
    z0 = jnp.dot(w0_ref[...], x_ref[...], preferred_element_type=jnp.float32)
    h0 = _relu_sin(z0 + b0_ref[...])
    z1 = jnp.dot(w1_ref[...], h0, preferred_element_type=jnp.float32)
    h1 = _relu_sin(z1 + b1_ref[...])
    z2 = jnp.dot(wh_ref[...], h1, preferred_element_type=jnp.float32)
    o_ref[...] = z2 + bh_ref[...]


def _round_bf16(a):
    u = jax.lax.bitcast_convert_type(a.astype(jnp.float32), jnp.uint32)
    u = (u + jnp.uint32(0x7FFF) + ((u >> 16) & jnp.uint32(1))) & jnp.uint32(0xFFFF0000)
    return jax.lax.bitcast_convert_type(u, jnp.float32)


@jax.jit
def _run(x, w0, b0, w1, b1, wh, bh):
    B, S, D = x.shape
    M = B * S

    w0f = (_OMEGA * w0).astype(jnp.float32).T
    b0c = _round_bf16(_OMEGA * b0).reshape(_H, 1)
    w1f = (_OMEGA * w1).astype(jnp.float32).T
    b1c = _round_bf16(_OMEGA * b1).reshape(_H, 1)
    whf = wh.astype(jnp.float32).T
    bhc = _round_bf16(bh).reshape(_OUT, 1)

    xt = x.reshape(M, D).T

    grid = (M // _TM,)
    out = pl.pallas_call(
        _fm_kernel,
        out_shape=jax.ShapeDtypeStruct((_OUT, M), jnp.float32),
        grid=grid,
        in_specs=[
            pl.BlockSpec((_IN, _TM), lambda i: (0, i)),
            pl.BlockSpec((_H, _IN), lambda i: (0, 0)),
            pl.BlockSpec((_H, 1), lambda i: (0, 0)),
            pl.BlockSpec((_H, _H), lambda i: (0, 0)),
            pl.BlockSpec((_H, 1), lambda i: (0, 0)),
            pl.BlockSpec((_OUT, _H), lambda i: (0, 0)),
            pl.BlockSpec((_OUT, 1), lambda i: (0, 0)),
        ],
        out_specs=pl.BlockSpec((_OUT, _TM), lambda i: (0, i)),
        compiler_params=pltpu.CompilerParams(
            dimension_semantics=("parallel",),
            vmem_limit_bytes=64 * 1024 * 1024,
        ),
        cost_estimate=pl.CostEstimate(
            flops=2 * M * ((_IN + 1) * _H + (_H + 1) * _H + (_H + 1) * _OUT),
            transcendentals=0,
            bytes_accessed=(_IN + _OUT) * 4 * M,
        ),
    )(xt, w0f, b0c, w1f, b1c, whf, bhc)

    return out.T.reshape(B, S, _OUT)


def kernel(x, w0, b0, w1, b1, wh, bh):
    return _run(x, w0, b0, w1, b1, wh, bh)


# speedup vs baseline: 10.2374x; 1.1765x over previous
import jax
import jax.numpy as jnp
from jax.experimental import pallas as pl
from jax.experimental.pallas import tpu as pltpu

_IN = 2
_H = 32
_OUT = 3
_OMEGA = 30.0
_TM = 32768

_INV_PI = 0.31830987334251404
_PI_F32 = 3.1415927410125732
_S1 = 0.9999992251396179
_S2 = -0.16665679216384888
_S3 = 0.008313223719596863
_S4 = -0.0001852341665653512


def _relu_sin(z):
    n = jnp.rint(z * _INV_PI)
    r = z - n * _PI_F32
    q = r * r
    p = _S3 + q * _S4
    p = _S2 + q * p
    p = _S1 + q * p
    s = r * p
    sb = jax.lax.shift_left(jnp.bitwise_and(n.astype(jnp.int32), 1), 31)
    s = jax.lax.bitcast_convert_type(
        jax.lax.bitcast_convert_type(s, jnp.int32) ^ sb, jnp.float32)
    return jnp.maximum(s, 0.0)


def _fm_kernel(x_ref, w0_ref, b0_ref, w1_ref, b1_ref, wh_ref, bh_ref, o_ref):
    z0 = jnp.dot(w0_ref[...], x_ref[...], preferred_element_type=jnp.float32)
    h0 = _relu_sin(z0 + b0_ref[...])
    z1 = jnp.dot(w1_ref[...], h0, preferred_element_type=jnp.float32)
    h1 = _relu_sin(z1 + b1_ref[...])
    z2 = jnp.dot(wh_ref[...], h1, preferred_element_type=jnp.float32)
    o_ref[...] = z2 + bh_ref[...]


def _round_bf16(a):
    u = jax.lax.bitcast_convert_type(a.astype(jnp.float32), jnp.uint32)
    u = (u + jnp.uint32(0x7FFF) + ((u >> 16) & jnp.uint32(1))) & jnp.uint32(0xFFFF0000)
    return jax.lax.bitcast_convert_type(u, jnp.float32)


@jax.jit
def _run(x, w0, b0, w1, b1, wh, bh):
    B, S, D = x.shape
    M = B * S

    w0f = (_OMEGA * w0).astype(jnp.float32).T
    b0c = _round_bf16(_OMEGA * b0).reshape(_H, 1)
    w1f = (_OMEGA * w1).astype(jnp.float32).T
    b1c = _round_bf16(_OMEGA * b1).reshape(_H, 1)
    whf = wh.astype(jnp.float32).T
    bhc = _round_bf16(bh).reshape(_OUT, 1)

    xt = x.reshape(M, D).T

    grid = (M // _TM,)
    out = pl.pallas_call(
        _fm_kernel,
        out_shape=jax.ShapeDtypeStruct((_OUT, M), jnp.float32),
        grid=grid,
        in_specs=[
            pl.BlockSpec((_IN, _TM), lambda i: (0, i)),
            pl.BlockSpec((_H, _IN), lambda i: (0, 0)),
            pl.BlockSpec((_H, 1), lambda i: (0, 0)),
            pl.BlockSpec((_H, _H), lambda i: (0, 0)),
            pl.BlockSpec((_H, 1), lambda i: (0, 0)),
            pl.BlockSpec((_OUT, _H), lambda i: (0, 0)),
            pl.BlockSpec((_OUT, 1), lambda i: (0, 0)),
        ],
        out_specs=pl.BlockSpec((_OUT, _TM), lambda i: (0, i)),
        compiler_params=pltpu.CompilerParams(
            dimension_semantics=("parallel",),
            vmem_limit_bytes=64 * 1024 * 1024,
        ),
        cost_estimate=pl.CostEstimate(
            flops=2 * M * ((_IN + 1) * _H + (_H + 1) * _H + (_H + 1) * _OUT),
            transcendentals=0,
            bytes_accessed=(_IN + _OUT) * 4 * M,
        ),
    )(xt, w0f, b0c, w1f, b1c, whf, bhc)

    return out.T.reshape(B, S, _OUT)


def kernel(x, w0, b0, w1, b1, wh, bh):
    return _run(x, w0, b0, w1, b1, wh, bh)


# speedup vs baseline: 11.1642x; 1.0905x over previous
import jax
import jax.numpy as jnp
from jax.experimental import pallas as pl
from jax.experimental.pallas import tpu as pltpu

_IN = 2
_H = 32
_OUT = 3
_OMEGA = 30.0
_TM = 32768

_INV_2PI = 0.15915493667125702
_TWO_PI = 6.2831854820251465
_C0 = 0.9999961256980896
_C1 = -0.1666470319032669
_C2 = 0.008317245170474052
_C3 = -0.00019376579439267516
_C4 = 2.1981200006848667e-06


def _relu_sin(z):
    n = jnp.rint(z * _INV_2PI)
    r = z - n * _TWO_PI
    q = r * r
    p = _C3 + q * _C4
    p = _C2 + q * p
    p = _C1 + q * p
    p = _C0 + q * p
    return jnp.maximum(r * p, 0.0)


def _fm_kernel(x_ref, w0_ref, b0_ref, w1_ref, b1_ref, wh_ref, bh_ref, o_ref):
    z0 = jnp.dot(w0_ref[...], x_ref[...], preferred_element_type=jnp.float32)
    h0 = _relu_sin(z0 + b0_ref[...])
    z1 = jnp.dot(w1_ref[...], h0, preferred_element_type=jnp.float32)
    h1 = _relu_sin(z1 + b1_ref[...])
    z2 = jnp.dot(wh_ref[...], h1, preferred_element_type=jnp.float32)
    o_ref[...] = z2 + bh_ref[...]


def _round_bf16(a):
    u = jax.lax.bitcast_convert_type(a.astype(jnp.float32), jnp.uint32)
    u = (u + jnp.uint32(0x7FFF) + ((u >> 16) & jnp.uint32(1))) & jnp.uint32(0xFFFF0000)
    return jax.lax.bitcast_convert_type(u, jnp.float32)


@jax.jit
def _run(x, w0, b0, w1, b1, wh, bh):
    B, S, D = x.shape
    M = B * S

    w0f = (_OMEGA * w0).astype(jnp.float32).T
    b0c = _round_bf16(_OMEGA * b0).reshape(_H, 1)
    w1f = (_OMEGA * w1).astype(jnp.float32).T
    b1c = _round_bf16(_OMEGA * b1).reshape(_H, 1)
    whf = wh.astype(jnp.float32).T
    bhc = _round_bf16(bh).reshape(_OUT, 1)

    xt = x.reshape(M, D).T

    grid = (M // _TM,)
    out = pl.pallas_call(
        _fm_kernel,
        out_shape=jax.ShapeDtypeStruct((_OUT, M), jnp.float32),
        grid=grid,
        in_specs=[
            pl.BlockSpec((_IN, _TM), lambda i: (0, i)),
            pl.BlockSpec((_H, _IN), lambda i: (0, 0)),
            pl.BlockSpec((_H, 1), lambda i: (0, 0)),
            pl.BlockSpec((_H, _H), lambda i: (0, 0)),
            pl.BlockSpec((_H, 1), lambda i: (0, 0)),
            pl.BlockSpec((_OUT, _H), lambda i: (0, 0)),
            pl.BlockSpec((_OUT, 1), lambda i: (0, 0)),
        ],
        out_specs=pl.BlockSpec((_OUT, _TM), lambda i: (0, i)),
        compiler_params=pltpu.CompilerParams(
            dimension_semantics=("parallel",),
            vmem_limit_bytes=64 * 1024 * 1024,
        ),
        cost_estimate=pl.CostEstimate(
            flops=2 * M * ((_IN + 1) * _H + (_H + 1) * _H + (_H + 1) * _OUT),
            transcendentals=0,
            bytes_accessed=(_IN + _OUT) * 4 * M,
        ),
    )(xt, w0f, b0c, w1f, b1c, whf, bhc)

    return out.T.reshape(B, S, _OUT)


def kernel(x, w0, b0, w1, b1, wh, bh):
    return _run(x, w0, b0, w1, b1, wh, bh)


# speedup vs baseline: 11.4600x; 1.0265x over previous
import jax
import jax.numpy as jnp
from jax.experimental import pallas as pl
from jax.experimental.pallas import tpu as pltpu

_IN = 2
_H = 32
_OUT = 3
_OMEGA = 30.0
_TM = 65536

_INV_2PI = 0.15915493667125702
_TWO_PI = 6.2831854820251465
_C0 = 0.9999961256980896
_C1 = -0.1666470319032669
_C2 = 0.008317245170474052
_C3 = -0.00019376579439267516
_C4 = 2.1981200006848667e-06


def _relu_sin(z):
    n = jnp.rint(z * _INV_2PI)
    r = z - n * _TWO_PI
    q = r * r
    p = _C3 + q * _C4
    p = _C2 + q * p
    p = _C1 + q * p
    p = _C0 + q * p
    return jnp.maximum(r * p, 0.0)


def _fm_kernel(x_ref, w0_ref, b0_ref, w1_ref, b1_ref, wh_ref, bh_ref, o_ref):
    z0 = jnp.dot(w0_ref[...], x_ref[...], preferred_element_type=jnp.float32)
    h0 = _relu_sin(z0 + b0_ref[...])
    z1 = jnp.dot(w1_ref[...], h0, preferred_element_type=jnp.float32)
    h1 = _relu_sin(z1 + b1_ref[...])
    z2 = jnp.dot(wh_ref[...], h1, preferred_element_type=jnp.float32)
    o_ref[...] = z2 + bh_ref[...]


def _round_bf16(a):
    u = jax.lax.bitcast_convert_type(a.astype(jnp.float32), jnp.uint32)
    u = (u + jnp.uint32(0x7FFF) + ((u >> 16) & jnp.uint32(1))) & jnp.uint32(0xFFFF0000)
    return jax.lax.bitcast_convert_type(u, jnp.float32)


@jax.jit
def _run(x, w0, b0, w1, b1, wh, bh):
    B, S, D = x.shape
    M = B * S

    w0f = (_OMEGA * w0).astype(jnp.float32).T
    b0c = _round_bf16(_OMEGA * b0).reshape(_H, 1)
    w1f = (_OMEGA * w1).astype(jnp.float32).T
    b1c = _round_bf16(_OMEGA * b1).reshape(_H, 1)
    whf = wh.astype(jnp.float32).T
    bhc = _round_bf16(bh).reshape(_OUT, 1)

    xt = x.reshape(M, D).T

    grid = (M // _TM,)
    out = pl.pallas_call(
        _fm_kernel,
        out_shape=jax.ShapeDtypeStruct((_OUT, M), jnp.float32),
        grid=grid,
        in_specs=[
            pl.BlockSpec((_IN, _TM), lambda i: (0, i)),
            pl.BlockSpec((_H, _IN), lambda i: (0, 0)),
            pl.BlockSpec((_H, 1), lambda i: (0, 0)),
            pl.BlockSpec((_H, _H), lambda i: (0, 0)),
            pl.BlockSpec((_H, 1), lambda i: (0, 0)),
            pl.BlockSpec((_OUT, _H), lambda i: (0, 0)),
            pl.BlockSpec((_OUT, 1), lambda i: (0, 0)),
        ],
        out_specs=pl.BlockSpec((_OUT, _TM), lambda i: (0, i)),
        compiler_params=pltpu.CompilerParams(
            dimension_semantics=("parallel",),
            vmem_limit_bytes=64 * 1024 * 1024,
        ),
        cost_estimate=pl.CostEstimate(
            flops=2 * M * ((_IN + 1) * _H + (_H + 1) * _H + (_H + 1) * _OUT),
            transcendentals=0,
            bytes_accessed=(_IN + _OUT) * 4 * M,
        ),
    )(xt, w0f, b0c, w1f, b1c, whf, bhc)

    return out.T.reshape(B, S, _OUT)


def kernel(x, w0, b0, w1, b1, wh, bh):
    return _run(x, w0, b0, w1, b1, wh, bh)
